# baseline (speedup 1.0000x reference)
import math

import jax
import jax.numpy as jnp
from jax import lax
from jax.experimental import pallas as pl
from jax.experimental.pallas import tpu as pltpu

_BB = 16


def _fused_kernel(x_ref, h_ref, qkt_ref, ws_ref, wv_ref, gmem_ref, gh_ref,
                  o_ref, *, eps):
    bb, seqlen, dim = x_ref.shape
    md, dsl = ws_ref.shape
    nw = (seqlen * dim) // dsl
    ms = qkt_ref.shape[1]

    xb = x_ref[...].astype(jnp.bfloat16).reshape(bb * nw, dsl)

    s = lax.dot_general(xb, ws_ref[...], (((1,), (1,)), ((), ())),
                        preferred_element_type=jnp.float32)
    sb = s.astype(jnp.bfloat16)

    sv = lax.dot_general(sb, wv_ref[...], (((1,), (1,)), ((), ())),
                         preferred_element_type=jnp.float32)

    t = jnp.dot(sb, qkt_ref[...], preferred_element_type=jnp.float32)

    t3 = t.reshape(bb, nw, ms)

    m = jnp.maximum(jnp.max(t3, axis=1, keepdims=True), 0.0)
    p = jnp.exp(t3 - m)
    denom = jnp.sum(p, axis=1, keepdims=True) + jnp.exp(-m)
    p = (p / denom).astype(jnp.bfloat16)

    sv3 = sv.astype(jnp.bfloat16).reshape(bb, nw, md)
    mem = lax.dot_general(p, sv3, (((1,), (1,)), ((0,), (0,))),
                          preferred_element_type=jnp.float32)

    ms1 = jnp.mean(mem * mem, axis=(1, 2), keepdims=True)
    hn = h_ref[...] + mem * lax.rsqrt(ms1 + eps) * gmem_ref[...]
    ms2 = jnp.mean(hn * hn, axis=(1, 2), keepdims=True)
    o_ref[...] = (hn * lax.rsqrt(ms2 + eps) * gh_ref[...]).astype(o_ref.dtype)


def kernel(x, h, q, ws_w, wk_w, wv_w, g_mem, g_h):
    B, seqlen, dim = x.shape
    _, memory_size, memory_dim = h.shape
    dsl = ws_w.shape[1]
    eps = float(jnp.finfo(jnp.float32).eps)
    scale = 1.0 / math.sqrt(memory_dim)

    ws_b = ws_w.astype(jnp.bfloat16)
    wv_b = wv_w.astype(jnp.bfloat16)
    qkt = (lax.dot_general(wk_w, q, (((0,), (1,)), ((), ())))
           * scale).astype(jnp.bfloat16)

    bb = _BB

    def xmap(i):
        return (i, 0, 0)

    def wmap(i):
        return (0, 0)

    return pl.pallas_call(
        lambda *refs: _fused_kernel(*refs, eps=eps),
        out_shape=jax.ShapeDtypeStruct((B, memory_size, memory_dim), h.dtype),
        grid=(B // bb,),
        in_specs=[
            pl.BlockSpec((bb, seqlen, dim), xmap),
            pl.BlockSpec((bb, memory_size, memory_dim), xmap),
            pl.BlockSpec((memory_dim, memory_size), wmap),
            pl.BlockSpec((memory_dim, dsl), wmap),
            pl.BlockSpec((memory_dim, memory_dim), wmap),
            pl.BlockSpec((memory_size, memory_dim), wmap),
            pl.BlockSpec((memory_size, memory_dim), wmap),
        ],
        out_specs=pl.BlockSpec((bb, memory_size, memory_dim), xmap),
        compiler_params=pltpu.CompilerParams(
            dimension_semantics=("parallel",),
            vmem_limit_bytes=50 * 1024 * 1024,
        ),
    )(x, h, qkt, ws_b, wv_b, g_mem, g_h)


# speedup vs baseline: 1.1294x; 1.1294x over previous
import math

import jax
import jax.numpy as jnp
from jax import lax
from jax.experimental import pallas as pl
from jax.experimental.pallas import tpu as pltpu

_BB = 64


def _fused_kernel(x_ref, h_ref, w_ref, gmem_ref, gh_ref,
                  o_ref, *, eps, dsl, ms):
    bb, seqlen, dim = x_ref.shape
    md = w_ref.shape[0]
    nw = (seqlen * dim) // dsl

    ws_b = w_ref[:, :dsl]
    wv_b = w_ref[:, dsl:dsl + md]
    qkt_b = w_ref[:, dsl + md:dsl + md + ms]

    xb = x_ref[...].astype(jnp.bfloat16).reshape(bb * nw, dsl)

    s = lax.dot_general(xb, ws_b, (((1,), (1,)), ((), ())),
                        preferred_element_type=jnp.float32)
    sb = s.astype(jnp.bfloat16)

    sv = lax.dot_general(sb, wv_b, (((1,), (1,)), ((), ())),
                         preferred_element_type=jnp.float32)

    t = jnp.dot(sb, qkt_b, preferred_element_type=jnp.float32)

    t3 = t.reshape(bb, nw, ms)

    m = jnp.maximum(jnp.max(t3, axis=1, keepdims=True), 0.0)
    p = jnp.exp(t3 - m)
    denom = jnp.sum(p, axis=1, keepdims=True) + jnp.exp(-m)
    p = (p / denom).astype(jnp.bfloat16)

    sv3 = sv.astype(jnp.bfloat16).reshape(bb, nw, md)
    mem = lax.dot_general(p, sv3, (((1,), (1,)), ((0,), (0,))),
                          preferred_element_type=jnp.float32)

    ms1 = jnp.mean(mem * mem, axis=(1, 2), keepdims=True)
    hn = h_ref[...] + mem * lax.rsqrt(ms1 + eps) * gmem_ref[...]
    ms2 = jnp.mean(hn * hn, axis=(1, 2), keepdims=True)
    o_ref[...] = (hn * lax.rsqrt(ms2 + eps) * gh_ref[...]).astype(o_ref.dtype)


def kernel(x, h, q, ws_w, wk_w, wv_w, g_mem, g_h):
    B, seqlen, dim = x.shape
    _, memory_size, memory_dim = h.shape
    dsl = ws_w.shape[1]
    eps = float(jnp.finfo(jnp.float32).eps)
    scale = 1.0 / math.sqrt(memory_dim)

    qkt = lax.dot_general(wk_w, q, (((0,), (1,)), ((), ()))) * scale
    w_all = jnp.concatenate([ws_w, wv_w, qkt], axis=1).astype(jnp.bfloat16)

    bb = _BB

    def xmap(i):
        return (i, 0, 0)

    def wmap(i):
        return (0, 0)

    wtot = dsl + memory_dim + memory_size
    return pl.pallas_call(
        lambda *refs: _fused_kernel(*refs, eps=eps, dsl=dsl, ms=memory_size),
        out_shape=jax.ShapeDtypeStruct((B, memory_size, memory_dim), h.dtype),
        grid=(B // bb,),
        in_specs=[
            pl.BlockSpec((bb, seqlen, dim), xmap),
            pl.BlockSpec((bb, memory_size, memory_dim), xmap),
            pl.BlockSpec((memory_dim, wtot), wmap),
            pl.BlockSpec((memory_size, memory_dim), wmap),
            pl.BlockSpec((memory_size, memory_dim), wmap),
        ],
        out_specs=pl.BlockSpec((bb, memory_size, memory_dim), xmap),
        compiler_params=pltpu.CompilerParams(
            dimension_semantics=("parallel",),
            vmem_limit_bytes=50 * 1024 * 1024,
        ),
    )(x, h, w_all, g_mem, g_h)
